# baseline (speedup 1.0000x reference)
import jax
import jax.numpy as jnp
from jax.experimental import pallas as pl


def _sum_kernel(x_ref, o_ref):
    o_ref[...] = jnp.sum(x_ref[...], axis=(0, 1), keepdims=True)


def kernel(pred_pos, faces, f_connectivity, f_connectivity_edges, f_area, bending_coeff):
    tri = pred_pos[faces]
    v0, v1, v2 = tri[:, 0], tri[:, 1], tri[:, 2]
    n = jnp.cross(v1 - v0, v2 - v0)
    fn = n / (jnp.linalg.norm(n, axis=-1, keepdims=True) + 1e-12)

    nn = fn[f_connectivity]
    n0, n1 = nn[:, 0], nn[:, 1]
    v = pred_pos[f_connectivity_edges]
    e = v[:, 1] - v[:, 0]
    l = jnp.linalg.norm(e, axis=-1, keepdims=True)
    e_norm = e / l
    a = f_area[:, 0][f_connectivity].sum(axis=-1)
    cos = (n0 * n1).sum(axis=-1)
    sin = (e_norm * jnp.cross(n0, n1)).sum(axis=-1)
    theta = jnp.arctan2(sin, cos)
    scale = l[..., 0] ** 2 / (4.0 * a)
    energy = bending_coeff[0] * scale * theta ** 2 / 2.0

    ne = energy.shape[0]
    pad = (-ne) % 1024
    epad = jnp.pad(energy, (0, pad)).reshape(-1, 128)
    out = pl.pallas_call(
        _sum_kernel,
        out_shape=jax.ShapeDtypeStruct((1, 1), jnp.float32),
    )(epad)
    return out[0, 0]


# speedup vs baseline: 10.3089x; 10.3089x over previous
import functools

import jax
import jax.numpy as jnp
from jax import lax
from jax.experimental import pallas as pl
from jax.experimental.pallas import tpu as pltpu, tpu_sc as plsc

NV = 50000
NF = 100000
NE = 150000

NC = 2
NS = 16
NW = NC * NS
LANES = 16
CHUNK = 128

FC = 32
FP = NW * CHUNK * FC
EC = 40
EP = NW * CHUNK * EC

_ATAN = (0.999998017, -0.333060167, 0.196054925, -0.122270662,
         0.0585597433, -0.0138876227)
PI = 3.14159265358979
PIO2 = PI / 2

_MESH = plsc.VectorSubcoreMesh(core_axis_name="c", subcore_axis_name="s")


def _rsqrt(x):
    i = lax.bitcast_convert_type(x, jnp.int32)
    i = jnp.int32(0x5F3759DF) - (i >> 1)
    y = lax.bitcast_convert_type(i, jnp.float32)
    for _ in range(3):
        y = y * (1.5 - 0.5 * x * y * y)
    return y


def _atan2_sq(s, d):
    p, q = jnp.abs(d), jnp.abs(s)
    mx, mn = jnp.maximum(p, q), jnp.minimum(p, q)
    z = mn / jnp.maximum(mx, 1e-30)
    z2 = z * z
    t = jnp.float32(_ATAN[5])
    for c in _ATAN[4::-1]:
        t = t * z2 + c
    t = t * z
    t = jnp.where(q > p, PIO2 - t, t)
    t = jnp.where(d < 0.0, PI - t, t)
    return t * t


def _wid():
    return lax.axis_index("s") * NC + lax.axis_index("c")


_F32B = pltpu.VMEM((CHUNK,), jnp.float32)


@functools.partial(
    pl.kernel,
    out_type=(jax.ShapeDtypeStruct((FP,), jnp.float32),) * 3,
    mesh=_MESH,
    scratch_types=[pltpu.VMEM((FC, CHUNK), jnp.int32)] * 3
    + [_F32B] * 9
    + [_F32B] * 3
    + [pltpu.SemaphoreType.DMA],
)
def _face_k(px, py, pz, f0, f1, f2, ocx, ocy, ocz,
            i0, i1, i2,
            b0x, b0y, b0z, b1x, b1y, b1z, b2x, b2y, b2z,
            obx, oby, obz, sem):
    w = _wid()
    rbase = w * FC
    fbase = w * FC * CHUNK
    pltpu.sync_copy(f0.at[pl.ds(rbase, FC)], i0)
    pltpu.sync_copy(f1.at[pl.ds(rbase, FC)], i1)
    pltpu.sync_copy(f2.at[pl.ds(rbase, FC)], i2)

    def chunk(j, carry):
        idx0, idx1, idx2 = i0.at[j], i1.at[j], i2.at[j]
        cps = [
            pltpu.async_copy(px.at[idx0], b0x, sem),
            pltpu.async_copy(py.at[idx0], b0y, sem),
            pltpu.async_copy(pz.at[idx0], b0z, sem),
            pltpu.async_copy(px.at[idx1], b1x, sem),
            pltpu.async_copy(py.at[idx1], b1y, sem),
            pltpu.async_copy(pz.at[idx1], b1z, sem),
            pltpu.async_copy(px.at[idx2], b2x, sem),
            pltpu.async_copy(py.at[idx2], b2y, sem),
            pltpu.async_copy(pz.at[idx2], b2z, sem),
        ]
        for cp in cps:
            cp.wait()
        for sb in range(8):
            sl = pl.ds(sb * LANES, LANES)
            ax, ay, az = b0x[sl], b0y[sl], b0z[sl]
            ux, uy, uz = b1x[sl] - ax, b1y[sl] - ay, b1z[sl] - az
            wx, wy, wz = b2x[sl] - ax, b2y[sl] - ay, b2z[sl] - az
            obx[sl] = uy * wz - uz * wy
            oby[sl] = uz * wx - ux * wz
            obz[sl] = ux * wy - uy * wx
        dst = pl.ds(fbase + j * CHUNK, CHUNK)
        pltpu.sync_copy(obx, ocx.at[dst])
        pltpu.sync_copy(oby, ocy.at[dst])
        pltpu.sync_copy(obz, ocz.at[dst])
        return carry

    lax.fori_loop(0, FC, chunk, 0)


@functools.partial(
    pl.kernel,
    out_type=jax.ShapeDtypeStruct((NW * LANES,), jnp.float32),
    mesh=_MESH,
    scratch_types=[pltpu.VMEM((EC, CHUNK), jnp.int32)] * 4
    + [_F32B] * 14
    + [pltpu.VMEM((LANES,), jnp.float32), pltpu.SemaphoreType.DMA],
)
def _edge_k(px, py, pz, cx, cy, cz, af, jc0, jc1, je0, je1, part_out,
            ic0, ic1, ie0, ie1,
            g0x, g0y, g0z, g0a, g1x, g1y, g1z, g1a,
            h0x, h0y, h0z, h1x, h1y, h1z,
            accb, sem):
    w = _wid()
    rbase = w * EC
    pltpu.sync_copy(jc0.at[pl.ds(rbase, EC)], ic0)
    pltpu.sync_copy(jc1.at[pl.ds(rbase, EC)], ic1)
    pltpu.sync_copy(je0.at[pl.ds(rbase, EC)], ie0)
    pltpu.sync_copy(je1.at[pl.ds(rbase, EC)], ie1)

    def chunk(j, acc):
        ix0, ix1, iv0, iv1 = ic0.at[j], ic1.at[j], ie0.at[j], ie1.at[j]
        cps = [
            pltpu.async_copy(cx.at[ix0], g0x, sem),
            pltpu.async_copy(cy.at[ix0], g0y, sem),
            pltpu.async_copy(cz.at[ix0], g0z, sem),
            pltpu.async_copy(af.at[ix0], g0a, sem),
            pltpu.async_copy(cx.at[ix1], g1x, sem),
            pltpu.async_copy(cy.at[ix1], g1y, sem),
            pltpu.async_copy(cz.at[ix1], g1z, sem),
            pltpu.async_copy(af.at[ix1], g1a, sem),
            pltpu.async_copy(px.at[iv0], h0x, sem),
            pltpu.async_copy(py.at[iv0], h0y, sem),
            pltpu.async_copy(pz.at[iv0], h0z, sem),
            pltpu.async_copy(px.at[iv1], h1x, sem),
            pltpu.async_copy(py.at[iv1], h1y, sem),
            pltpu.async_copy(pz.at[iv1], h1z, sem),
        ]
        for cp in cps:
            cp.wait()
        for sb in range(8):
            sl = pl.ds(sb * LANES, LANES)
            c0x, c0y, c0z, a0 = g0x[sl], g0y[sl], g0z[sl], g0a[sl]
            c1x, c1y, c1z, a1 = g1x[sl], g1y[sl], g1z[sl], g1a[sl]
            ex = h1x[sl] - h0x[sl]
            ey = h1y[sl] - h0y[sl]
            ez = h1z[sl] - h0z[sl]
            ll = ex * ex + ey * ey + ez * ez
            dot = c0x * c1x + c0y * c1y + c0z * c1z
            gx = c0y * c1z - c0z * c1y
            gy = c0z * c1x - c0x * c1z
            gz = c0x * c1y - c0y * c1x
            sn = ex * gx + ey * gy + ez * gz
            s = sn * _rsqrt(ll)
            th2 = _atan2_sq(s, dot)
            acc = acc + th2 * ll / (4.0 * (a0 + a1))
        return acc

    acc = lax.fori_loop(0, EC, chunk, jnp.zeros((LANES,), jnp.float32))
    accb[...] = acc
    pltpu.sync_copy(accb, part_out.at[pl.ds(w * LANES, LANES)])


def kernel(pred_pos, faces, f_connectivity, f_connectivity_edges, f_area,
           bending_coeff):
    px, py, pz = pred_pos[:, 0], pred_pos[:, 1], pred_pos[:, 2]
    ft = jnp.pad(faces.T.astype(jnp.int32), ((0, 0), (0, FP - NF)))
    ft = ft.reshape(3, FP // CHUNK, CHUNK)
    af = jnp.pad(f_area[:, 0], (0, FP - NF))
    fct = jnp.pad(f_connectivity.T.astype(jnp.int32), ((0, 0), (0, EP - NE)))
    fct = fct.reshape(2, EP // CHUNK, CHUNK)
    fet = jnp.pad(f_connectivity_edges.T.astype(jnp.int32),
                  ((0, 0), (0, EP - NE)))
    fet = fet.reshape(2, EP // CHUNK, CHUNK)

    cx, cy, cz = _face_k(px, py, pz, ft[0], ft[1], ft[2])
    parts = _edge_k(px, py, pz, cx, cy, cz, af,
                    fct[0], fct[1], fet[0], fet[1])
    return jnp.sum(parts) * (bending_coeff[0] * 0.5)
